# baseline (speedup 1.0000x reference)
import functools

import jax
import jax.numpy as jnp
from jax import lax
from jax.experimental import pallas as pl
from jax.experimental.pallas import tpu as pltpu
from jax.experimental.pallas import tpu_sc as plsc

EMBED_DIM = 64
MAX_LEN = 200
SCALE = float(EMBED_DIM) ** 0.5

NUM_CORES = 2
NUM_SUBCORES = 16
NW = NUM_CORES * NUM_SUBCORES

CHUNK = 128


def _make_sc_kernel(total, vocab):
    assert total % (NW * CHUNK) == 0
    b_per_w = total // NW
    nchunk = b_per_w // CHUNK
    mesh = plsc.VectorSubcoreMesh(
        core_axis_name="c",
        subcore_axis_name="s",
        num_cores=NUM_CORES,
        num_subcores=NUM_SUBCORES,
    )

    @functools.partial(
        pl.kernel,
        out_type=jax.ShapeDtypeStruct((total, EMBED_DIM), jnp.float32),
        mesh=mesh,
        compiler_params=pltpu.CompilerParams(use_tc_tiling_on_sc=False),
        scratch_types=[
            pltpu.VMEM((CHUNK,), jnp.int32),
            pltpu.VMEM((CHUNK, EMBED_DIM), jnp.float32),
            pltpu.VMEM((MAX_LEN, EMBED_DIM), jnp.float32),
            pltpu.SemaphoreType.DMA,
        ],
    )
    def sc_kernel(ids_hbm, table_hbm, pos_hbm, out_hbm, idx_v, rows_v, pos_v, gsem):
        wid = lax.axis_index("c") * NUM_SUBCORES + lax.axis_index("s")
        wbase = wid * b_per_w
        pltpu.sync_copy(pos_hbm, pos_v)

        def chunk_body(i, _):
            base = wbase + i * CHUNK
            pltpu.sync_copy(ids_hbm.at[pl.ds(base, CHUNK)], idx_v)
            pltpu.async_copy(table_hbm.at[idx_v], rows_v, gsem).wait()
            start = lax.rem(i * CHUNK, MAX_LEN)

            def row_body(r, _):
                p = lax.rem(start + r, MAX_LEN)
                for g in range(EMBED_DIM // 16):
                    sl = pl.ds(g * 16, 16)
                    rows_v[r, sl] = rows_v[r, sl] * SCALE + pos_v[p, sl]
                return 0

            lax.fori_loop(0, CHUNK, row_body, 0)
            pltpu.sync_copy(rows_v, out_hbm.at[pl.ds(base, CHUNK)])
            return 0

        lax.fori_loop(0, nchunk, chunk_body, 0)

    return sc_kernel


def kernel(item_id, item_table, pos_table):
    batch, max_len = item_id.shape
    assert max_len == MAX_LEN
    total = batch * max_len
    ids_flat = item_id.reshape(total)
    sc = _make_sc_kernel(total, item_table.shape[0])
    out = sc(ids_flat, item_table, pos_table)
    return out.reshape(batch, max_len, EMBED_DIM)


# speedup vs baseline: 1.1876x; 1.1876x over previous
import functools

import jax
import jax.numpy as jnp
from jax import lax
from jax.experimental import pallas as pl
from jax.experimental.pallas import tpu as pltpu
from jax.experimental.pallas import tpu_sc as plsc

EMBED_DIM = 64
MAX_LEN = 200
SCALE = float(EMBED_DIM) ** 0.5

NUM_CORES = 2
NUM_SUBCORES = 16
NW = NUM_CORES * NUM_SUBCORES

GATHER = 128
CHUNK = 512
SUBG = CHUNK // GATHER


def _make_sc_kernel(total):
    assert total % (NW * CHUNK) == 0
    b_per_w = total // NW
    nchunk = b_per_w // CHUNK
    mesh = plsc.VectorSubcoreMesh(
        core_axis_name="c",
        subcore_axis_name="s",
        num_cores=NUM_CORES,
        num_subcores=NUM_SUBCORES,
    )

    @functools.partial(
        pl.kernel,
        out_type=jax.ShapeDtypeStruct((total, EMBED_DIM), jnp.float32),
        mesh=mesh,
        compiler_params=pltpu.CompilerParams(use_tc_tiling_on_sc=False),
        scratch_types=[
            pltpu.VMEM((b_per_w,), jnp.int32),
            pltpu.VMEM((2, CHUNK, EMBED_DIM), jnp.float32),
            pltpu.VMEM((MAX_LEN, EMBED_DIM), jnp.float32),
            pltpu.SemaphoreType.DMA,
            pltpu.SemaphoreType.DMA,
            pltpu.SemaphoreType.DMA,
            pltpu.SemaphoreType.DMA,
        ],
    )
    def sc_kernel(ids_hbm, table_hbm, pos_hbm, out_hbm,
                  idx_v, rows_v, pos_v, gsem0, gsem1, wsem0, wsem1):
        wid = lax.axis_index("c") * NUM_SUBCORES + lax.axis_index("s")
        wbase = wid * b_per_w
        pltpu.sync_copy(pos_hbm, pos_v)
        pltpu.sync_copy(ids_hbm.at[pl.ds(wbase, b_per_w)], idx_v)
        gsems = (gsem0, gsem1)
        wsems = (wsem0, wsem1)

        def fire_gathers(i, b, sem):
            for j in range(SUBG):
                pltpu.async_copy(
                    table_hbm.at[idx_v.at[pl.ds(i * CHUNK + j * GATHER, GATHER)]],
                    rows_v.at[b, pl.ds(j * GATHER, GATHER)],
                    sem,
                )

        def wait_gathers(i, b, sem):
            for j in range(SUBG):
                pltpu.make_async_copy(
                    table_hbm.at[idx_v.at[pl.ds(i * CHUNK + j * GATHER, GATHER)]],
                    rows_v.at[b, pl.ds(j * GATHER, GATHER)],
                    sem,
                ).wait()

        def wait_writeback(i, b, sem):
            pltpu.make_async_copy(
                rows_v.at[b],
                out_hbm.at[pl.ds(wbase + i * CHUNK, CHUNK)],
                sem,
            ).wait()

        fire_gathers(0, 0, gsems[0])

        @pl.loop(0, nchunk, step=2)
        def pipeline(i2):
            for b in range(2):
                i = i2 + b
                nb = 1 - b
                sem_b = gsems[b]

                @pl.when(i + 1 < nchunk)
                def _prefetch():
                    @pl.when(i >= 1)
                    def _():
                        wait_writeback(i - 1, nb, wsems[nb])

                    fire_gathers(i + 1, nb, gsems[nb])

                wait_gathers(i, b, sem_b)
                start = lax.rem(i * CHUNK, MAX_LEN)

                def row_body(r, _):
                    p = lax.rem(start + r, MAX_LEN)
                    for g in range(EMBED_DIM // 16):
                        sl = pl.ds(g * 16, 16)
                        rows_v[b, r, sl] = rows_v[b, r, sl] * SCALE + pos_v[p, sl]
                    return 0

                lax.fori_loop(0, CHUNK, row_body, 0)
                pltpu.async_copy(
                    rows_v.at[b],
                    out_hbm.at[pl.ds(wbase + i * CHUNK, CHUNK)],
                    wsems[b],
                )

        wait_writeback(nchunk - 2, 0, wsems[0])
        wait_writeback(nchunk - 1, 1, wsems[1])

    return sc_kernel


def kernel(item_id, item_table, pos_table):
    batch, max_len = item_id.shape
    assert max_len == MAX_LEN
    total = batch * max_len
    ids_flat = item_id.reshape(total)
    sc = _make_sc_kernel(total)
    out = sc(ids_flat, item_table, pos_table)
    return out.reshape(batch, max_len, EMBED_DIM)
